# Initial kernel scaffold; baseline (speedup 1.0000x reference)
import functools

import jax
import jax.numpy as jnp
from jax import lax
from jax.experimental import pallas as pl
from jax.experimental.pallas import tpu as pltpu
from jax.experimental.pallas import tpu_sc as plsc

N_NODES = 10000
D = 128
NC = 2
NS = 16
NW = NC * NS
G = 128
ROWS_PER_TILE = N_NODES // NS
ZCHUNK = 125
LANES = 16
DSL = D // LANES


def _mm_body(x_ref, w_ref, o_ref):
    o_ref[...] = jnp.dot(x_ref[...], w_ref[...],
                         preferred_element_type=jnp.float32)


def _matmul(x, w):
    m = x.shape[0]
    bm = 500
    grid = m // bm
    return pl.pallas_call(
        _mm_body,
        grid=(grid,),
        in_specs=[
            pl.BlockSpec((bm, D), lambda i: (i, 0)),
            pl.BlockSpec((D, D), lambda i: (0, 0)),
        ],
        out_specs=pl.BlockSpec((bm, D), lambda i: (i, 0)),
        out_shape=jax.ShapeDtypeStruct((m, D), jnp.float32),
    )(x, w)


def _spmm_body(ngroups):
    def body(support_hbm, src_hbm, dst_hbm, val_hbm, out_hbm,
             src_v, dst_v, val_v, rows_v, zbuf_v, acc_sh, sem):
        cid = lax.axis_index("c")
        sid = lax.axis_index("s")
        wid = sid * NC + cid

        zeros16 = jnp.zeros((LANES,), jnp.float32)

        @pl.loop(0, ZCHUNK)
        def _(i):
            for j in range(DSL):
                zbuf_v[i, pl.ds(j * LANES, LANES)] = zeros16

        for k in range(ROWS_PER_TILE // ZCHUNK):
            pltpu.sync_copy(
                zbuf_v,
                acc_sh.at[pl.ds(sid * ROWS_PER_TILE + k * ZCHUNK, ZCHUNK)])
        plsc.subcore_barrier()

        pltpu.sync_copy(src_hbm.at[wid], src_v)
        pltpu.sync_copy(dst_hbm.at[wid], dst_v)
        pltpu.sync_copy(val_hbm.at[wid], val_v)

        @pl.loop(0, ngroups)
        def _(g):
            pltpu.async_copy(support_hbm.at[src_v.at[g]], rows_v, sem).wait()

            @pl.loop(0, G)
            def _(e):
                v = val_v[g, e]
                for j in range(DSL):
                    sl = pl.ds(j * LANES, LANES)
                    rows_v[e, sl] = rows_v[e, sl] * v

            pltpu.sync_copy(rows_v, acc_sh.at[dst_v.at[g]], add=True)

        plsc.subcore_barrier()

        for k in range(ROWS_PER_TILE // ZCHUNK):
            r0 = sid * ROWS_PER_TILE + k * ZCHUNK
            pltpu.sync_copy(acc_sh.at[pl.ds(r0, ZCHUNK)],
                            out_hbm.at[cid, pl.ds(r0, ZCHUNK)])

    return body


def _spmm(support, src, dst, val, ngroups):
    mesh = plsc.VectorSubcoreMesh(core_axis_name="c", subcore_axis_name="s",
                                  num_cores=NC, num_subcores=NS)
    f = pl.kernel(
        _spmm_body(ngroups),
        out_type=jax.ShapeDtypeStruct((NC, N_NODES, D), jnp.float32),
        mesh=mesh,
        scratch_types=[
            pltpu.VMEM((ngroups, G), jnp.int32),
            pltpu.VMEM((ngroups, G), jnp.int32),
            pltpu.VMEM((ngroups, G), jnp.float32),
            pltpu.VMEM((G, D), jnp.float32),
            pltpu.VMEM((ZCHUNK, D), jnp.float32),
            pltpu.VMEM_SHARED((N_NODES, D), jnp.float32),
            pltpu.SemaphoreType.DMA,
        ],
    )
    return f(support, src, dst, val)


def _combine_body(p_ref, b_ref, o_ref):
    o_ref[...] = p_ref[0] + p_ref[1] + b_ref[...]


def _combine(partials, bias):
    bm = 500
    grid = N_NODES // bm
    return pl.pallas_call(
        _combine_body,
        grid=(grid,),
        in_specs=[
            pl.BlockSpec((NC, bm, D), lambda i: (0, i, 0)),
            pl.BlockSpec((1, D), lambda i: (0, 0)),
        ],
        out_specs=pl.BlockSpec((bm, D), lambda i: (i, 0)),
        out_shape=jax.ShapeDtypeStruct((N_NODES, D), jnp.float32),
    )(partials, bias.reshape(1, D))


def kernel(sparse_poly_edge_index, sparse_poly_values, input_feature, weight,
           bias):
    support = _matmul(input_feature, weight)

    src = sparse_poly_edge_index[1].astype(jnp.int32)
    dst = sparse_poly_edge_index[0].astype(jnp.int32)
    val = sparse_poly_values

    n_edges = src.shape[0]
    epw = -(-n_edges // (NW * G)) * G
    pad = NW * epw - n_edges
    if pad:
        src = jnp.concatenate([src, jnp.zeros((pad,), jnp.int32)])
        dst = jnp.concatenate([dst, jnp.zeros((pad,), jnp.int32)])
        val = jnp.concatenate([val, jnp.zeros((pad,), jnp.float32)])
    ngroups = epw // G
    src = src.reshape(NW, ngroups, G)
    dst = dst.reshape(NW, ngroups, G)
    val = val.reshape(NW, ngroups, G)

    partials = _spmm(support, src, dst, val, ngroups)
    return _combine(partials, bias)


# speedup vs baseline: 3.0774x; 3.0774x over previous
import functools

import jax
import jax.numpy as jnp
from jax import lax
from jax.experimental import pallas as pl
from jax.experimental.pallas import tpu as pltpu
from jax.experimental.pallas import tpu_sc as plsc

N_NODES = 10000
D = 128
NC = 2
NS = 16
NW = NC * NS
G = 128
CH = 8
ACC_ROWS = 10240
ROWS_PER_TILE = ACC_ROWS // NS
ZCHUNK = 128
LANES = 16
DSL = D // LANES


def _mm_body(x_ref, w_ref, o_ref):
    o_ref[...] = jnp.dot(x_ref[...], w_ref[...],
                         preferred_element_type=jnp.float32)


def _matmul(x, w):
    m = x.shape[0]
    bm = 1000
    grid = m // bm
    return pl.pallas_call(
        _mm_body,
        grid=(grid,),
        in_specs=[
            pl.BlockSpec((bm, D), lambda i: (i, 0)),
            pl.BlockSpec((D, D), lambda i: (0, 0)),
        ],
        out_specs=pl.BlockSpec((bm, D), lambda i: (i, 0)),
        out_shape=jax.ShapeDtypeStruct((m, D), jnp.float32),
    )(x, w)


def _spmm_body(nchunks):
    def body(support_hbm, src_hbm, dst_hbm, val_hbm, out_hbm,
             src_c, dst_c, val_c, rows_v, acc_sh, sem):
        cid = lax.axis_index("c")
        sid = lax.axis_index("s")
        wid = sid * NC + cid

        zeros16 = jnp.zeros((LANES,), jnp.float32)

        @pl.loop(0, ZCHUNK)
        def _(i):
            for j in range(DSL):
                rows_v[i, pl.ds(j * LANES, LANES)] = zeros16

        for k in range(ROWS_PER_TILE // ZCHUNK):
            pltpu.sync_copy(
                rows_v,
                acc_sh.at[pl.ds(sid * ROWS_PER_TILE + k * ZCHUNK, ZCHUNK)])
        plsc.subcore_barrier()

        @pl.loop(0, nchunks)
        def _(c):
            pltpu.sync_copy(src_hbm.at[wid, pl.ds(c * CH, CH)], src_c)
            pltpu.sync_copy(dst_hbm.at[wid, pl.ds(c * CH, CH)], dst_c)
            pltpu.sync_copy(val_hbm.at[wid, pl.ds(c * CH, CH)], val_c)

            @pl.loop(0, CH)
            def _(g):
                pltpu.async_copy(support_hbm.at[src_c.at[g]], rows_v,
                                 sem).wait()

                @pl.loop(0, G // LANES)
                def _(s):
                    vv = val_c[g, pl.ds(s * LANES, LANES)]
                    for l in range(LANES):
                        v = vv[l]
                        e = s * LANES + l
                        for j in range(DSL):
                            sl = pl.ds(j * LANES, LANES)
                            rows_v[e, sl] = rows_v[e, sl] * v

                pltpu.sync_copy(rows_v, acc_sh.at[dst_c.at[g]], add=True)

        plsc.subcore_barrier()

        for k in range(ROWS_PER_TILE // ZCHUNK):
            r0 = sid * ROWS_PER_TILE + k * ZCHUNK
            if (15 * ROWS_PER_TILE + k * ZCHUNK + ZCHUNK) <= N_NODES:
                pltpu.sync_copy(acc_sh.at[pl.ds(r0, ZCHUNK)],
                                out_hbm.at[cid, pl.ds(r0, ZCHUNK)])
            else:
                @pl.when(sid < NS - 1)
                def _(r0=r0):
                    pltpu.sync_copy(acc_sh.at[pl.ds(r0, ZCHUNK)],
                                    out_hbm.at[cid, pl.ds(r0, ZCHUNK)])

        @pl.when(sid == NS - 1)
        def _():
            base = 15 * ROWS_PER_TILE
            done = ((N_NODES - base) // ZCHUNK) * ZCHUNK
            rem = N_NODES - base - done
            pltpu.sync_copy(acc_sh.at[pl.ds(base + done, rem)],
                            out_hbm.at[cid, pl.ds(base + done, rem)])

    return body


def _spmm(support, src, dst, val, nchunks):
    mesh = plsc.VectorSubcoreMesh(core_axis_name="c", subcore_axis_name="s",
                                  num_cores=NC, num_subcores=NS)
    f = pl.kernel(
        _spmm_body(nchunks),
        out_type=jax.ShapeDtypeStruct((NC, N_NODES, D), jnp.float32),
        mesh=mesh,
        scratch_types=[
            pltpu.VMEM((CH, G), jnp.int32),
            pltpu.VMEM((CH, G), jnp.int32),
            pltpu.VMEM((CH, G), jnp.float32),
            pltpu.VMEM((G, D), jnp.float32),
            pltpu.VMEM_SHARED((ACC_ROWS, D), jnp.float32),
            pltpu.SemaphoreType.DMA,
        ],
    )
    return f(support, src, dst, val)


def _combine_body(p_ref, b_ref, o_ref):
    o_ref[...] = p_ref[0] + p_ref[1] + b_ref[...]


def _combine(partials, bias):
    bm = 1000
    grid = N_NODES // bm
    return pl.pallas_call(
        _combine_body,
        grid=(grid,),
        in_specs=[
            pl.BlockSpec((NC, bm, D), lambda i: (0, i, 0)),
            pl.BlockSpec((1, D), lambda i: (0, 0)),
        ],
        out_specs=pl.BlockSpec((bm, D), lambda i: (i, 0)),
        out_shape=jax.ShapeDtypeStruct((N_NODES, D), jnp.float32),
    )(partials, bias.reshape(1, D))


def kernel(sparse_poly_edge_index, sparse_poly_values, input_feature, weight,
           bias):
    support = _matmul(input_feature, weight)

    src = sparse_poly_edge_index[1].astype(jnp.int32)
    dst = sparse_poly_edge_index[0].astype(jnp.int32)
    val = sparse_poly_values

    n_edges = src.shape[0]
    epw = -(-n_edges // (NW * G * CH)) * G * CH
    pad = NW * epw - n_edges
    if pad:
        src = jnp.concatenate([src, jnp.zeros((pad,), jnp.int32)])
        dst = jnp.concatenate([dst, jnp.zeros((pad,), jnp.int32)])
        val = jnp.concatenate([val, jnp.zeros((pad,), jnp.float32)])
    ngroups = epw // G
    src = src.reshape(NW, ngroups, G)
    dst = dst.reshape(NW, ngroups, G)
    val = val.reshape(NW, ngroups, G)

    partials = _spmm(support, src, dst, val, ngroups // CH)
    return _combine(partials, bias)


# speedup vs baseline: 3.5960x; 1.1685x over previous
import functools

import jax
import jax.numpy as jnp
from jax import lax
from jax.experimental import pallas as pl
from jax.experimental.pallas import tpu as pltpu
from jax.experimental.pallas import tpu_sc as plsc

N_NODES = 10000
D = 128
NC = 2
NS = 16
NW = NC * NS
G = 128
CH = 16
ACC_ROWS = 10240
ROWS_PER_TILE = ACC_ROWS // NS
ZCHUNK = 128
LANES = 16
DSL = D // LANES


def _mm_body(x_ref, w_ref, o_ref):
    o_ref[...] = jnp.dot(x_ref[...], w_ref[...],
                         preferred_element_type=jnp.float32)


def _matmul(x, w):
    m = x.shape[0]
    bm = 1000
    grid = m // bm
    return pl.pallas_call(
        _mm_body,
        grid=(grid,),
        in_specs=[
            pl.BlockSpec((bm, D), lambda i: (i, 0)),
            pl.BlockSpec((D, D), lambda i: (0, 0)),
        ],
        out_specs=pl.BlockSpec((bm, D), lambda i: (i, 0)),
        out_shape=jax.ShapeDtypeStruct((m, D), jnp.float32),
    )(x, w)


def _spmm_body(nchunks):
    def body(support_hbm, src_hbm, dst_hbm, val_hbm, out_hbm,
             src_c, dst_c, val_c, rows_v, acc_sh, sem0, sem1):
        sems = (sem0, sem1)
        cid = lax.axis_index("c")
        sid = lax.axis_index("s")
        wid = sid * NC + cid

        zeros16 = jnp.zeros((LANES,), jnp.float32)

        @pl.loop(0, ZCHUNK)
        def _(i):
            for j in range(DSL):
                rows_v[0, i, pl.ds(j * LANES, LANES)] = zeros16

        for k in range(ROWS_PER_TILE // ZCHUNK):
            pltpu.sync_copy(
                rows_v.at[0],
                acc_sh.at[pl.ds(sid * ROWS_PER_TILE + k * ZCHUNK, ZCHUNK)])
        plsc.subcore_barrier()

        @pl.loop(0, nchunks)
        def _(c):
            pltpu.sync_copy(src_hbm.at[wid, pl.ds(c * CH, CH)], src_c)
            pltpu.sync_copy(dst_hbm.at[wid, pl.ds(c * CH, CH)], dst_c)
            pltpu.sync_copy(val_hbm.at[wid, pl.ds(c * CH, CH)], val_c)

            def start_gather(g, b):
                return pltpu.async_copy(support_hbm.at[src_c.at[g]],
                                        rows_v.at[b], sems[b])

            descs = [None, None]
            descs[0] = start_gather(0, 0)
            for g in range(CH):
                b = g & 1
                descs[b].wait()
                if g + 1 < CH:
                    descs[(g + 1) & 1] = start_gather(g + 1, (g + 1) & 1)

                @pl.loop(0, G // LANES)
                def _(s, g=g, b=b):
                    vv = val_c[g, pl.ds(s * LANES, LANES)]
                    for l in range(LANES):
                        v = vv[l]
                        e = s * LANES + l
                        for j in range(DSL):
                            sl = pl.ds(j * LANES, LANES)
                            rows_v[b, e, sl] = rows_v[b, e, sl] * v

                pltpu.sync_copy(rows_v.at[b], acc_sh.at[dst_c.at[g]],
                                add=True)

        plsc.subcore_barrier()

        for k in range(ROWS_PER_TILE // ZCHUNK):
            r0 = sid * ROWS_PER_TILE + k * ZCHUNK
            if (15 * ROWS_PER_TILE + k * ZCHUNK + ZCHUNK) <= N_NODES:
                pltpu.sync_copy(acc_sh.at[pl.ds(r0, ZCHUNK)],
                                out_hbm.at[cid, pl.ds(r0, ZCHUNK)])
            else:
                @pl.when(sid < NS - 1)
                def _(r0=r0):
                    pltpu.sync_copy(acc_sh.at[pl.ds(r0, ZCHUNK)],
                                    out_hbm.at[cid, pl.ds(r0, ZCHUNK)])

        @pl.when(sid == NS - 1)
        def _():
            base = 15 * ROWS_PER_TILE
            done = ((N_NODES - base) // ZCHUNK) * ZCHUNK
            rem = N_NODES - base - done
            pltpu.sync_copy(acc_sh.at[pl.ds(base + done, rem)],
                            out_hbm.at[cid, pl.ds(base + done, rem)])

    return body


def _spmm(support, src, dst, val, nchunks):
    mesh = plsc.VectorSubcoreMesh(core_axis_name="c", subcore_axis_name="s",
                                  num_cores=NC, num_subcores=NS)
    f = pl.kernel(
        _spmm_body(nchunks),
        out_type=jax.ShapeDtypeStruct((NC, N_NODES, D), jnp.float32),
        mesh=mesh,
        scratch_types=[
            pltpu.VMEM((CH, G), jnp.int32),
            pltpu.VMEM((CH, G), jnp.int32),
            pltpu.VMEM((CH, G), jnp.float32),
            pltpu.VMEM((2, G, D), jnp.float32),
            pltpu.VMEM_SHARED((ACC_ROWS, D), jnp.float32),
            pltpu.SemaphoreType.DMA,
            pltpu.SemaphoreType.DMA,
        ],
    )
    return f(support, src, dst, val)


def _combine_body(p_ref, b_ref, o_ref):
    o_ref[...] = p_ref[0] + p_ref[1] + b_ref[...]


def _combine(partials, bias):
    bm = 1000
    grid = N_NODES // bm
    return pl.pallas_call(
        _combine_body,
        grid=(grid,),
        in_specs=[
            pl.BlockSpec((NC, bm, D), lambda i: (0, i, 0)),
            pl.BlockSpec((1, D), lambda i: (0, 0)),
        ],
        out_specs=pl.BlockSpec((bm, D), lambda i: (i, 0)),
        out_shape=jax.ShapeDtypeStruct((N_NODES, D), jnp.float32),
    )(partials, bias.reshape(1, D))


def kernel(sparse_poly_edge_index, sparse_poly_values, input_feature, weight,
           bias):
    support = _matmul(input_feature, weight)

    src = sparse_poly_edge_index[1].astype(jnp.int32)
    dst = sparse_poly_edge_index[0].astype(jnp.int32)
    val = sparse_poly_values

    n_edges = src.shape[0]
    epw = -(-n_edges // (NW * G * CH)) * G * CH
    pad = NW * epw - n_edges
    if pad:
        src = jnp.concatenate([src, jnp.zeros((pad,), jnp.int32)])
        dst = jnp.concatenate([dst, jnp.zeros((pad,), jnp.int32)])
        val = jnp.concatenate([val, jnp.zeros((pad,), jnp.float32)])
    ngroups = epw // G
    src = src.reshape(NW, ngroups, G)
    dst = dst.reshape(NW, ngroups, G)
    val = val.reshape(NW, ngroups, G)

    partials = _spmm(support, src, dst, val, ngroups // CH)
    return _combine(partials, bias)
